# baseline (speedup 1.0000x reference)
import jax
import jax.numpy as jnp
from jax.experimental import pallas as pl


def _assign_body(x_ref, cm2_ref, cn_ref, out_ref):
    xb = x_ref[...].reshape(-1, x_ref.shape[-1])
    s = jnp.dot(xb, cm2_ref[...], preferred_element_type=jnp.float32)
    xnorm = jnp.sum(xb * xb, axis=1, keepdims=True)
    dist = (xnorm + s) + cn_ref[...]

    R = dist.shape[0]
    v = dist[:, 0:128]
    j = jnp.zeros((R, 128), jnp.float32)
    for c in (1, 2, 3):
        vc = dist[:, c * 128:(c + 1) * 128]
        jc = jnp.full((R, 128), float(c * 128), jnp.float32)
        take = vc < v
        v = jnp.where(take, vc, v)
        j = jnp.where(take, jc, j)
    lane = jax.lax.broadcasted_iota(jnp.int32, (R, 128), 1).astype(jnp.float32)
    j = j + lane
    vt = v.T
    jt = j.T
    n = 128
    while n > 1:
        h = n // 2
        va, vb = vt[:h], vt[h:n]
        ja, jb = jt[:h], jt[h:n]
        take_b = (vb < va) | ((vb == va) & (jb < ja))
        vt = jnp.where(take_b, vb, va)
        jt = jnp.where(take_b, jb, ja)
        n = h
    out_ref[...] = jt[0].astype(jnp.int32)


def kernel(x, C, Cnorm):
    B, T, D = x.shape
    K = C.shape[1]
    cm2 = C * (-2.0)
    bb = 2
    out = pl.pallas_call(
        _assign_body,
        grid=(B // bb,),
        in_specs=[
            pl.BlockSpec((bb, T, D), lambda i: (i, 0, 0)),
            pl.BlockSpec((D, K), lambda i: (0, 0)),
            pl.BlockSpec((1, K), lambda i: (0, 0)),
        ],
        out_specs=pl.BlockSpec((bb * T,), lambda i: (i,)),
        out_shape=jax.ShapeDtypeStruct((B * T,), jnp.int32),
    )(x, cm2, Cnorm)
    return out.reshape(B, T)
